# Initial kernel scaffold; baseline (speedup 1.0000x reference)
import jax
import jax.numpy as jnp
from jax.experimental import pallas as pl


def kernel(x, edge_index, edge_attr, u, batch, W1, b1, W2, b2):
    raise NotImplementedError("write your pallas kernel here")


# speedup vs baseline: 4.5223x; 4.5223x over previous
import functools

import jax
import jax.numpy as jnp
from jax import lax
from jax.experimental import pallas as pl
from jax.experimental.pallas import tpu as pltpu
from jax.experimental.pallas import tpu_sc as plsc

N = 10000
D = 128
B = 256
NC = 2
NS = 16
NW = NC * NS
CH = 320
PIECE = 64
NPIECE = CH // PIECE
CW = 16


def _sc_segment_sum(x, batch_i32):
    mesh = plsc.VectorSubcoreMesh(core_axis_name="c", subcore_axis_name="s")

    @functools.partial(
        pl.kernel,
        out_type=(
            jax.ShapeDtypeStruct((NC, B, D), jnp.float32),
            jax.ShapeDtypeStruct((NC, B, CW), jnp.float32),
        ),
        mesh=mesh,
        scratch_types=[
            pltpu.VMEM((CH, D), jnp.float32),
            pltpu.VMEM((NPIECE, PIECE), jnp.int32),
            pltpu.VMEM((16,), jnp.int32),
            pltpu.VMEM((CH, CW), jnp.float32),
            pltpu.VMEM((NS, D), jnp.float32),
            pltpu.VMEM((NS, CW), jnp.float32),
            pltpu.VMEM_SHARED((B, D), jnp.float32),
            pltpu.VMEM_SHARED((B, CW), jnp.float32),
        ],
    )
    def seg_sum(x_hbm, batch_hbm, sums_hbm, cnts_hbm,
                xrows, idx2d, idx16, ones_v, zsum, zcnt, acc, acc_cnt):
        c = lax.axis_index("c")
        s = lax.axis_index("s")
        wid = s * NC + c

        zero16 = jnp.zeros((16,), jnp.float32)
        one16 = jnp.ones((16,), jnp.float32)
        for r in range(NS):
            for k in range(D // 16):
                zsum[r, pl.ds(k * 16, 16)] = zero16
            zcnt[r, :] = zero16
        for r in range(CH):
            ones_v[r, :] = one16

        pltpu.sync_copy(zsum, acc.at[pl.ds(s * NS, NS)])
        pltpu.sync_copy(zcnt, acc_cnt.at[pl.ds(s * NS, NS)])
        plsc.subcore_barrier()

        base = pl.multiple_of(wid * CH, CH)

        @pl.when(wid < NW - 1)
        def _full():
            pltpu.sync_copy(x_hbm.at[pl.ds(base, CH)], xrows)
            for p in range(NPIECE):
                pltpu.sync_copy(batch_hbm.at[pl.ds(base + p * PIECE, PIECE)],
                                idx2d.at[p])
            for p in range(NPIECE):
                pltpu.sync_copy(xrows.at[pl.ds(p * PIECE, PIECE)],
                                acc.at[idx2d.at[p]], add=True)
                pltpu.sync_copy(ones_v.at[pl.ds(p * PIECE, PIECE)],
                                acc_cnt.at[idx2d.at[p]], add=True)

        @pl.when(wid == NW - 1)
        def _tail():
            pltpu.sync_copy(x_hbm.at[pl.ds(N - 80, 80)], xrows.at[pl.ds(0, 80)])
            pltpu.sync_copy(batch_hbm.at[pl.ds(N - 80, PIECE)], idx2d.at[0])
            pltpu.sync_copy(batch_hbm.at[pl.ds(N - 16, 16)], idx16)
            pltpu.sync_copy(xrows.at[pl.ds(0, PIECE)],
                            acc.at[idx2d.at[0]], add=True)
            pltpu.sync_copy(ones_v.at[pl.ds(0, PIECE)],
                            acc_cnt.at[idx2d.at[0]], add=True)
            pltpu.sync_copy(xrows.at[pl.ds(PIECE, 16)], acc.at[idx16], add=True)
            pltpu.sync_copy(ones_v.at[pl.ds(0, 16)], acc_cnt.at[idx16],
                            add=True)

        plsc.subcore_barrier()

        pltpu.sync_copy(acc.at[pl.ds(s * NS, NS)],
                        sums_hbm.at[c, pl.ds(s * NS, NS)])
        pltpu.sync_copy(acc_cnt.at[pl.ds(s * NS, NS)],
                        cnts_hbm.at[c, pl.ds(s * NS, NS)])

    return seg_sum(x, batch_i32)


def _mlp(sums2, cnts2, u, w1u, w1a, b1, w2, b2):
    def body(sums_ref, cnts_ref, u_ref, w1u_ref, w1a_ref, b1_ref, w2_ref,
             b2_ref, o_ref):
        sums = sums_ref[0] + sums_ref[1]
        cnt = cnts_ref[0] + cnts_ref[1]
        inv = 1.0 / jnp.maximum(cnt[:, 0:1], 1.0)
        agg = sums * inv
        h = jnp.dot(u_ref[...], w1u_ref[...], preferred_element_type=jnp.float32)
        h = h + jnp.dot(agg, w1a_ref[...], preferred_element_type=jnp.float32)
        h = jnp.maximum(h + b1_ref[...], 0.0)
        o_ref[...] = (jnp.dot(h, w2_ref[...], preferred_element_type=jnp.float32)
                      + b2_ref[...])

    return pl.pallas_call(
        body,
        out_shape=jax.ShapeDtypeStruct((B, w2.shape[1]), jnp.float32),
    )(sums2, cnts2, u, w1u, w1a, b1.reshape(1, -1), w2, b2.reshape(1, -1))


def kernel(x, edge_index, edge_attr, u, batch, W1, b1, W2, b2):
    del edge_index, edge_attr
    batch32 = batch.astype(jnp.int32)
    sums2, cnts2 = _sc_segment_sum(x, batch32)
    return _mlp(sums2, cnts2, u, W1[:D], W1[D:], b1, W2, b2)
